# baseline (speedup 1.0000x reference)
import functools

import jax
import jax.numpy as jnp
from jax.experimental import pallas as pl
from jax.experimental.pallas import tpu as pltpu

COST_CLASS, COST_BBOX, COST_GIOU = 1.0, 5.0, 2.0
ALPHA, GAMMA = 0.25, 2.0
W = 8


def _matcher_kernel(lg_ref, bq_ref, ids_ref, gt_ref, oi_ref, oj_ref, *scr,
                    nr, nc):
    f32 = jnp.float32
    i32 = jnp.int32
    INF = f32(jnp.inf)
    cost_ref = scr[0]
    v_refs = scr[1:1 + W]
    c4rv_refs = scr[1 + W:1 + 2 * W]
    spc_refs = scr[1 + 2 * W:1 + 3 * W]
    path_refs = scr[1 + 3 * W:1 + 4 * W]
    rem_refs = scr[1 + 4 * W:1 + 5 * W]
    u_s, r4c_s, c4r_s, visr_s, vism_s = scr[1 + 5 * W:]
    iota_c = jax.lax.broadcasted_iota(i32, (1, nc), 1)
    iota_r = jax.lax.broadcasted_iota(i32, (1, nr), 1)

    p = jax.nn.sigmoid(lg_ref[:])
    neg = (1.0 - ALPHA) * (p * p) * (-jnp.log(1.0 - p))
    pos = ALPHA * ((1.0 - p) * (1.0 - p)) * (-jnp.log(p + 1e-8))
    diff = pos - neg
    ids = ids_ref[0]
    iota_cls = jax.lax.broadcasted_iota(i32, (W * nr, lg_ref.shape[0]), 1)
    onehot = (iota_cls == ids).astype(f32)
    c_cls = jnp.dot(onehot, diff, preferred_element_type=f32,
                    precision=jax.lax.Precision.HIGHEST)

    g_cx = gt_ref[0, :, 0:1]
    g_cy = gt_ref[0, :, 1:2]
    g_w = gt_ref[0, :, 2:3]
    g_h = gt_ref[0, :, 3:4]
    q_cx = bq_ref[0:1, :]
    q_cy = bq_ref[1:2, :]
    q_w = bq_ref[2:3, :]
    q_h = bq_ref[3:4, :]

    l1 = ((jnp.abs(g_cx - q_cx) + jnp.abs(g_cy - q_cy))
          + jnp.abs(g_w - q_w)) + jnp.abs(g_h - q_h)

    g_x0 = g_cx - 0.5 * g_w
    g_y0 = g_cy - 0.5 * g_h
    g_x1 = g_cx + 0.5 * g_w
    g_y1 = g_cy + 0.5 * g_h
    q_x0 = q_cx - 0.5 * q_w
    q_y0 = q_cy - 0.5 * q_h
    q_x1 = q_cx + 0.5 * q_w
    q_y1 = q_cy + 0.5 * q_h
    area_g = (g_x1 - g_x0) * (g_y1 - g_y0)
    area_q = (q_x1 - q_x0) * (q_y1 - q_y0)
    whx = jnp.maximum(jnp.minimum(g_x1, q_x1) - jnp.maximum(g_x0, q_x0), 0.0)
    why = jnp.maximum(jnp.minimum(g_y1, q_y1) - jnp.maximum(g_y0, q_y0), 0.0)
    inter = whx * why
    union = area_q + area_g - inter
    iou = inter / union
    ex = jnp.maximum(jnp.maximum(g_x1, q_x1) - jnp.minimum(g_x0, q_x0), 0.0)
    ey = jnp.maximum(jnp.maximum(g_y1, q_y1) - jnp.minimum(g_y0, q_y0), 0.0)
    enclose = ex * ey
    giou = iou - (enclose - union) / enclose
    cost_ref[:] = (COST_CLASS * c_cls + COST_BBOX * l1) + COST_GIOU * (-giou)

    for w in range(W):
        v_refs[w][:] = jnp.zeros((1, nc), f32)
        c4rv_refs[w][:] = jnp.full((1, nr), -1, i32)

    def init_col(t, carry):
        for w in range(W):
            r4c_s[w, t] = i32(-1)
        return carry

    jax.lax.fori_loop(0, nc + 1, init_col, 0)

    def init_row(t, carry):
        for w in range(W):
            u_s[w, t] = f32(0.0)
            c4r_s[w, t] = i32(-1)
        return carry

    jax.lax.fori_loop(0, nr + 1, init_row, 0)

    def outer(cur_row, carry):
        for w in range(W):
            spc_refs[w][:] = jnp.full((1, nc), INF, f32)
            path_refs[w][:] = jnp.full((1, nc), -1, i32)
            rem_refs[w][:] = jnp.ones((1, nc), i32)

        def cond(st):
            alld = st[0][0]
            for w in range(1, W):
                alld = alld & st[0][w]
            return jnp.logical_not(alld)

        def body(st):
            done, i, mv, sink, k = st
            ndone, ni, nmv, nsink, nk = [], [], [], [], []
            for w in range(W):
                crow = cost_ref[pl.ds(w * nr + i[w], 1), :]
                u_i = u_s[w, i[w]]
                r = ((mv[w] + crow) - u_i) - v_refs[w][:]
                rem = rem_refs[w][:] != 0
                spc = spc_refs[w][:]
                better = rem & (r < spc) & jnp.logical_not(done[w])
                spc = jnp.where(better, r, spc)
                spc_refs[w][:] = spc
                path_refs[w][:] = jnp.where(better, i[w], path_refs[w][:])
                masked = jnp.where(rem, spc, INF)
                mv2 = jnp.min(masked)
                j = jnp.min(jnp.where(masked == mv2, iota_c, nc))
                rem_refs[w][:] = jnp.where(
                    (iota_c == j) & jnp.logical_not(done[w]), 0,
                    rem_refs[w][:])
                r4cj = r4c_s[w, j]
                unmatched = r4cj < 0
                ndone.append(done[w] | unmatched)
                nsink.append(jnp.where(done[w], sink[w],
                                       jnp.where(unmatched, j, sink[w])))
                ni.append(jnp.where(done[w] | unmatched, i[w], r4cj))
                nmv.append(jnp.where(done[w], mv[w], mv2))
                visr_s[w, k[w]] = r4cj
                vism_s[w, k[w]] = mv2
                nk.append(jnp.where(done[w] | unmatched, k[w], k[w] + 1))
            return tuple(ndone), tuple(ni), tuple(nmv), tuple(nsink), tuple(nk)

        init = (tuple(jnp.asarray(False) for _ in range(W)),
                tuple(i32(cur_row) for _ in range(W)),
                tuple(f32(0.0) for _ in range(W)),
                tuple(i32(-1) for _ in range(W)),
                tuple(i32(0) for _ in range(W)))
        _, _, mvf, sink, kf = jax.lax.while_loop(cond, body, init)

        for w in range(W):
            u_s[w, cur_row] = u_s[w, cur_row] + mvf[w]

        for w in range(W):
            def dual(t, carry, w=w):
                row = visr_s[w, t]
                u_s[w, row] = u_s[w, row] + (mvf[w] - vism_s[w, t])
                return carry

            jax.lax.fori_loop(0, kf[w], dual, 0)
            sc = rem_refs[w][:] == 0
            v_refs[w][:] = v_refs[w][:] - jnp.where(
                sc, mvf[w] - spc_refs[w][:], 0.0)

        def acond(st):
            alld = st[0][0]
            for w in range(1, W):
                alld = alld & st[0][w]
            return jnp.logical_not(alld)

        def abody(st):
            done, j = st
            ndone, nj = [], []
            for w in range(W):
                pi = jnp.sum(jnp.where(iota_c == j[w], path_refs[w][:], 0))
                jw = jnp.where(done[w], nc, j[w])
                piw = jnp.where(done[w], nr, pi)
                r4c_s[w, jw] = pi
                jn = c4r_s[w, piw]
                c4r_s[w, piw] = j[w]
                c4rv_refs[w][:] = jnp.where(
                    iota_r == piw, j[w], c4rv_refs[w][:])
                ndone.append(done[w] | (pi == cur_row))
                nj.append(jnp.where(done[w], j[w], jn))
            return tuple(ndone), tuple(nj)

        ainit = (tuple(jnp.asarray(False) for _ in range(W)), sink)
        jax.lax.while_loop(acond, abody, ainit)
        return carry

    jax.lax.fori_loop(0, nr, outer, 0)

    iota_sub = jax.lax.broadcasted_iota(i32, (nr, 1), 0)
    for w in range(W):
        c4r = c4rv_refs[w][:]
        c4r_col = c4r.reshape(nr, 1)
        rank = jnp.sum((c4r < c4r_col).astype(i32), axis=1, keepdims=True)
        oh = rank == iota_r
        oj_ref[0, 0, w * nr:(w + 1) * nr] = jnp.sum(
            jnp.where(oh, iota_sub, 0), axis=0)
        oi_ref[0, 0, w * nr:(w + 1) * nr] = jnp.sum(
            jnp.where(oh, c4r_col, 0), axis=0)


def kernel(pred_logits, pred_boxes, tgt_labels, tgt_boxes):
    bs, nq, ncls = pred_logits.shape
    ngt = tgt_labels.shape[1]
    ncls_pad = ((ncls + 7) // 8) * 8
    ng = bs // W

    lg = jnp.zeros((ncls_pad, nq), jnp.float32).at[:ncls].set(pred_logits[0].T)
    bq = jnp.zeros((8, nq), jnp.float32).at[:4].set(pred_boxes[0].T)
    ids3 = tgt_labels.reshape(ng, W * ngt, 1).astype(jnp.int32)
    gt3 = tgt_boxes.reshape(ng, W * ngt, 4)

    body = functools.partial(_matcher_kernel, nr=ngt, nc=nq)
    scratch = [pltpu.VMEM((W * ngt, nq), jnp.float32)]
    scratch += [pltpu.VMEM((1, nq), jnp.float32) for _ in range(W)]
    scratch += [pltpu.VMEM((1, ngt), jnp.int32) for _ in range(W)]
    scratch += [pltpu.VMEM((1, nq), jnp.float32) for _ in range(W)]
    scratch += [pltpu.VMEM((1, nq), jnp.int32) for _ in range(W)]
    scratch += [pltpu.VMEM((1, nq), jnp.int32) for _ in range(W)]
    scratch += [
        pltpu.SMEM((W, ngt + 1), jnp.float32),
        pltpu.SMEM((W, nq + 1), jnp.int32),
        pltpu.SMEM((W, ngt + 1), jnp.int32),
        pltpu.SMEM((W, ngt + 2), jnp.int32),
        pltpu.SMEM((W, ngt + 2), jnp.float32),
    ]
    oi, oj = pl.pallas_call(
        body,
        grid=(ng,),
        in_specs=[
            pl.BlockSpec((ncls_pad, nq), lambda b: (0, 0)),
            pl.BlockSpec((8, nq), lambda b: (0, 0)),
            pl.BlockSpec((1, W * ngt, 1), lambda b: (b, 0, 0)),
            pl.BlockSpec((1, W * ngt, 4), lambda b: (b, 0, 0)),
        ],
        out_specs=[
            pl.BlockSpec((1, 1, W * ngt), lambda b: (b, 0, 0)),
            pl.BlockSpec((1, 1, W * ngt), lambda b: (b, 0, 0)),
        ],
        out_shape=[
            jax.ShapeDtypeStruct((ng, 1, W * ngt), jnp.int32),
            jax.ShapeDtypeStruct((ng, 1, W * ngt), jnp.int32),
        ],
        scratch_shapes=scratch,
        compiler_params=pltpu.CompilerParams(
            dimension_semantics=("parallel",),
        ),
    )(lg, bq, ids3, gt3)
    return oi.reshape(bs, ngt), oj.reshape(bs, ngt)


# speedup vs baseline: 1.6488x; 1.6488x over previous
import functools

import jax
import jax.numpy as jnp
from jax.experimental import pallas as pl
from jax.experimental.pallas import tpu as pltpu

COST_CLASS, COST_BBOX, COST_GIOU = 1.0, 5.0, 2.0
ALPHA, GAMMA = 0.25, 2.0
W = 4


def _matcher_kernel(lg_ref, bq_ref, ids_ref, gt_ref, oi_ref, oj_ref,
                    cost_ref, v_ref, c4rv_ref, spc_ref, path_ref, rem_ref,
                    u_s, r4c_s, c4r_s, visr_s, vism_s, *, nr, nc):
    f32 = jnp.float32
    i32 = jnp.int32
    INF = f32(jnp.inf)
    iota_c = jax.lax.broadcasted_iota(i32, (W, nc), 1)
    iota_r = jax.lax.broadcasted_iota(i32, (W, nr), 1)
    iota_r1 = jax.lax.broadcasted_iota(i32, (1, nr), 1)

    def pack(vals, dtype):
        return jnp.concatenate(
            [jnp.reshape(v, (1, 1)).astype(dtype) for v in vals], axis=0)

    p = jax.nn.sigmoid(lg_ref[:])
    neg = (1.0 - ALPHA) * (p * p) * (-jnp.log(1.0 - p))
    pos = ALPHA * ((1.0 - p) * (1.0 - p)) * (-jnp.log(p + 1e-8))
    diff = pos - neg
    ids = ids_ref[0]
    iota_cls = jax.lax.broadcasted_iota(i32, (W * nr, lg_ref.shape[0]), 1)
    onehot = (iota_cls == ids).astype(f32)
    c_cls = jnp.dot(onehot, diff, preferred_element_type=f32,
                    precision=jax.lax.Precision.HIGHEST)

    g_cx = gt_ref[0, :, 0:1]
    g_cy = gt_ref[0, :, 1:2]
    g_w = gt_ref[0, :, 2:3]
    g_h = gt_ref[0, :, 3:4]
    q_cx = bq_ref[0:1, :]
    q_cy = bq_ref[1:2, :]
    q_w = bq_ref[2:3, :]
    q_h = bq_ref[3:4, :]

    l1 = ((jnp.abs(g_cx - q_cx) + jnp.abs(g_cy - q_cy))
          + jnp.abs(g_w - q_w)) + jnp.abs(g_h - q_h)

    g_x0 = g_cx - 0.5 * g_w
    g_y0 = g_cy - 0.5 * g_h
    g_x1 = g_cx + 0.5 * g_w
    g_y1 = g_cy + 0.5 * g_h
    q_x0 = q_cx - 0.5 * q_w
    q_y0 = q_cy - 0.5 * q_h
    q_x1 = q_cx + 0.5 * q_w
    q_y1 = q_cy + 0.5 * q_h
    area_g = (g_x1 - g_x0) * (g_y1 - g_y0)
    area_q = (q_x1 - q_x0) * (q_y1 - q_y0)
    whx = jnp.maximum(jnp.minimum(g_x1, q_x1) - jnp.maximum(g_x0, q_x0), 0.0)
    why = jnp.maximum(jnp.minimum(g_y1, q_y1) - jnp.maximum(g_y0, q_y0), 0.0)
    inter = whx * why
    union = area_q + area_g - inter
    iou = inter / union
    ex = jnp.maximum(jnp.maximum(g_x1, q_x1) - jnp.minimum(g_x0, q_x0), 0.0)
    ey = jnp.maximum(jnp.maximum(g_y1, q_y1) - jnp.minimum(g_y0, q_y0), 0.0)
    enclose = ex * ey
    giou = iou - (enclose - union) / enclose
    cost_ref[:] = (COST_CLASS * c_cls + COST_BBOX * l1) + COST_GIOU * (-giou)

    v_ref[:] = jnp.zeros((W, nc), f32)
    c4rv_ref[:] = jnp.full((W, nr), -1, i32)

    def init_col(t, carry):
        for w in range(W):
            r4c_s[w, t] = i32(-1)
        return carry

    jax.lax.fori_loop(0, nc + 1, init_col, 0)

    def init_row(t, carry):
        for w in range(W):
            u_s[w, t] = f32(0.0)
            c4r_s[w, t] = i32(-1)
        return carry

    jax.lax.fori_loop(0, nr + 1, init_row, 0)

    def outer(cur_row, carry):
        spc_ref[:] = jnp.full((W, nc), INF, f32)
        path_ref[:] = jnp.full((W, nc), -1, i32)
        rem_ref[:] = jnp.ones((W, nc), i32)

        def cond(st):
            alld = st[0][0]
            for w in range(1, W):
                alld = alld & st[0][w]
            return jnp.logical_not(alld)

        def body(st):
            done, i, mv_col, sink, k = st
            crow = jnp.concatenate(
                [cost_ref[pl.ds(w * nr + i[w], 1), :] for w in range(W)],
                axis=0)
            u_col = pack([u_s[w, i[w]] for w in range(W)], f32)
            i_col = pack(i, i32)
            act_col = pack([jnp.logical_not(d) for d in done], i32) != 0
            r = ((mv_col + crow) - u_col) - v_ref[:]
            rem = rem_ref[:] != 0
            spc = spc_ref[:]
            better = rem & (r < spc) & act_col
            spc = jnp.where(better, r, spc)
            spc_ref[:] = spc
            path_ref[:] = jnp.where(better, i_col, path_ref[:])
            masked = jnp.where(rem, spc, INF)
            mv2_col = jnp.min(masked, axis=1, keepdims=True)
            j_col = jnp.min(jnp.where(masked == mv2_col, iota_c, nc),
                            axis=1, keepdims=True)
            rem_ref[:] = jnp.where((iota_c == j_col) & act_col, 0,
                                   rem_ref[:])
            ndone, ni, nsink, nk = [], [], [], []
            for w in range(W):
                j_w = j_col[w, 0]
                mv2_w = mv2_col[w, 0]
                r4cj = r4c_s[w, j_w]
                unmatched = r4cj < 0
                ndone.append(done[w] | unmatched)
                nsink.append(jnp.where(done[w], sink[w],
                                       jnp.where(unmatched, j_w, sink[w])))
                ni.append(jnp.where(done[w] | unmatched, i[w], r4cj))
                visr_s[w, k[w]] = r4cj
                vism_s[w, k[w]] = mv2_w
                nk.append(jnp.where(done[w] | unmatched, k[w], k[w] + 1))
            nmv_col = jnp.where(pack(done, i32) != 0, mv_col, mv2_col)
            return tuple(ndone), tuple(ni), nmv_col, tuple(nsink), tuple(nk)

        init = (tuple(jnp.asarray(False) for _ in range(W)),
                tuple(i32(cur_row) for _ in range(W)),
                jnp.zeros((W, 1), f32),
                tuple(i32(-1) for _ in range(W)),
                tuple(i32(0) for _ in range(W)))
        _, _, mvf_col, sink, kf = jax.lax.while_loop(cond, body, init)

        for w in range(W):
            mvf_w = mvf_col[w, 0]
            u_s[w, cur_row] = u_s[w, cur_row] + mvf_w

            def dual(t, carry, w=w, mvf_w=mvf_w):
                row = visr_s[w, t]
                u_s[w, row] = u_s[w, row] + (mvf_w - vism_s[w, t])
                return carry

            jax.lax.fori_loop(0, kf[w], dual, 0)

        sc = rem_ref[:] == 0
        v_ref[:] = v_ref[:] - jnp.where(sc, mvf_col - spc_ref[:], 0.0)

        def acond(st):
            alld = st[0][0]
            for w in range(1, W):
                alld = alld & st[0][w]
            return jnp.logical_not(alld)

        def abody(st):
            done, j = st
            j_col = pack(j, i32)
            act_col = pack([jnp.logical_not(d) for d in done], i32) != 0
            pi_col = jnp.sum(jnp.where((iota_c == j_col) & act_col,
                                       path_ref[:], 0),
                             axis=1, keepdims=True)
            c4rv_ref[:] = jnp.where((iota_r == pi_col) & act_col,
                                    j_col, c4rv_ref[:])
            ndone, nj = [], []
            for w in range(W):
                pi = pi_col[w, 0]
                jw = jnp.where(done[w], nc, j[w])
                piw = jnp.where(done[w], nr, pi)
                r4c_s[w, jw] = pi
                jn = c4r_s[w, piw]
                c4r_s[w, piw] = j[w]
                ndone.append(done[w] | (pi == cur_row))
                nj.append(jnp.where(done[w], j[w], jn))
            return tuple(ndone), tuple(nj)

        ainit = (tuple(jnp.asarray(False) for _ in range(W)), sink)
        jax.lax.while_loop(acond, abody, ainit)
        return carry

    jax.lax.fori_loop(0, nr, outer, 0)

    iota_sub = jax.lax.broadcasted_iota(i32, (nr, 1), 0)
    for w in range(W):
        c4r = c4rv_ref[pl.ds(w, 1), :]
        c4r_col = c4r.reshape(nr, 1)
        rank = jnp.sum((c4r < c4r_col).astype(i32), axis=1, keepdims=True)
        oh = rank == iota_r1
        oj_ref[0, 0, w * nr:(w + 1) * nr] = jnp.sum(
            jnp.where(oh, iota_sub, 0), axis=0)
        oi_ref[0, 0, w * nr:(w + 1) * nr] = jnp.sum(
            jnp.where(oh, c4r_col, 0), axis=0)


def kernel(pred_logits, pred_boxes, tgt_labels, tgt_boxes):
    bs, nq, ncls = pred_logits.shape
    ngt = tgt_labels.shape[1]
    ncls_pad = ((ncls + 7) // 8) * 8
    ng = bs // W

    lg = jnp.zeros((ncls_pad, nq), jnp.float32).at[:ncls].set(pred_logits[0].T)
    bq = jnp.zeros((8, nq), jnp.float32).at[:4].set(pred_boxes[0].T)
    ids3 = tgt_labels.reshape(ng, W * ngt, 1).astype(jnp.int32)
    gt3 = tgt_boxes.reshape(ng, W * ngt, 4)

    body = functools.partial(_matcher_kernel, nr=ngt, nc=nq)
    scratch = [
        pltpu.VMEM((W * ngt, nq), jnp.float32),
        pltpu.VMEM((W, nq), jnp.float32),
        pltpu.VMEM((W, ngt), jnp.int32),
        pltpu.VMEM((W, nq), jnp.float32),
        pltpu.VMEM((W, nq), jnp.int32),
        pltpu.VMEM((W, nq), jnp.int32),
        pltpu.SMEM((W, ngt + 1), jnp.float32),
        pltpu.SMEM((W, nq + 1), jnp.int32),
        pltpu.SMEM((W, ngt + 1), jnp.int32),
        pltpu.SMEM((W, ngt + 2), jnp.int32),
        pltpu.SMEM((W, ngt + 2), jnp.float32),
    ]
    oi, oj = pl.pallas_call(
        body,
        grid=(ng,),
        in_specs=[
            pl.BlockSpec((ncls_pad, nq), lambda b: (0, 0)),
            pl.BlockSpec((8, nq), lambda b: (0, 0)),
            pl.BlockSpec((1, W * ngt, 1), lambda b: (b, 0, 0)),
            pl.BlockSpec((1, W * ngt, 4), lambda b: (b, 0, 0)),
        ],
        out_specs=[
            pl.BlockSpec((1, 1, W * ngt), lambda b: (b, 0, 0)),
            pl.BlockSpec((1, 1, W * ngt), lambda b: (b, 0, 0)),
        ],
        out_shape=[
            jax.ShapeDtypeStruct((ng, 1, W * ngt), jnp.int32),
            jax.ShapeDtypeStruct((ng, 1, W * ngt), jnp.int32),
        ],
        scratch_shapes=scratch,
        compiler_params=pltpu.CompilerParams(
            dimension_semantics=("parallel",),
        ),
    )(lg, bq, ids3, gt3)
    return oi.reshape(bs, ngt), oj.reshape(bs, ngt)


# speedup vs baseline: 2.2220x; 1.3477x over previous
import functools

import jax
import jax.numpy as jnp
from jax.experimental import pallas as pl
from jax.experimental.pallas import tpu as pltpu

COST_CLASS, COST_BBOX, COST_GIOU = 1.0, 5.0, 2.0
ALPHA, GAMMA = 0.25, 2.0
W = 8


def _matcher_kernel(lg_ref, bq_ref, ids_ref, gt_ref, oi_ref, oj_ref,
                    cost_ref, v_ref, c4rv_ref, spc_ref, path_ref, rem_ref,
                    u_s, r4c_s, c4r_s, visr_s, vism_s, *, nr, nc):
    f32 = jnp.float32
    i32 = jnp.int32
    INF = f32(jnp.inf)
    iota_c = jax.lax.broadcasted_iota(i32, (W, nc), 1)
    iota_r = jax.lax.broadcasted_iota(i32, (W, nr), 1)
    iota_r1 = jax.lax.broadcasted_iota(i32, (1, nr), 1)

    def pack(vals, dtype):
        return jnp.concatenate(
            [jnp.reshape(v, (1, 1)).astype(dtype) for v in vals], axis=0)

    p = jax.nn.sigmoid(lg_ref[:])
    neg = (1.0 - ALPHA) * (p * p) * (-jnp.log(1.0 - p))
    pos = ALPHA * ((1.0 - p) * (1.0 - p)) * (-jnp.log(p + 1e-8))
    diff = pos - neg
    ids = ids_ref[0]
    iota_cls = jax.lax.broadcasted_iota(i32, (W * nr, lg_ref.shape[0]), 1)
    onehot = (iota_cls == ids).astype(f32)
    c_cls = jnp.dot(onehot, diff, preferred_element_type=f32,
                    precision=jax.lax.Precision.HIGHEST)

    g_cx = gt_ref[0, :, 0:1]
    g_cy = gt_ref[0, :, 1:2]
    g_w = gt_ref[0, :, 2:3]
    g_h = gt_ref[0, :, 3:4]
    q_cx = bq_ref[0:1, :]
    q_cy = bq_ref[1:2, :]
    q_w = bq_ref[2:3, :]
    q_h = bq_ref[3:4, :]

    l1 = ((jnp.abs(g_cx - q_cx) + jnp.abs(g_cy - q_cy))
          + jnp.abs(g_w - q_w)) + jnp.abs(g_h - q_h)

    g_x0 = g_cx - 0.5 * g_w
    g_y0 = g_cy - 0.5 * g_h
    g_x1 = g_cx + 0.5 * g_w
    g_y1 = g_cy + 0.5 * g_h
    q_x0 = q_cx - 0.5 * q_w
    q_y0 = q_cy - 0.5 * q_h
    q_x1 = q_cx + 0.5 * q_w
    q_y1 = q_cy + 0.5 * q_h
    area_g = (g_x1 - g_x0) * (g_y1 - g_y0)
    area_q = (q_x1 - q_x0) * (q_y1 - q_y0)
    whx = jnp.maximum(jnp.minimum(g_x1, q_x1) - jnp.maximum(g_x0, q_x0), 0.0)
    why = jnp.maximum(jnp.minimum(g_y1, q_y1) - jnp.maximum(g_y0, q_y0), 0.0)
    inter = whx * why
    union = area_q + area_g - inter
    iou = inter / union
    ex = jnp.maximum(jnp.maximum(g_x1, q_x1) - jnp.minimum(g_x0, q_x0), 0.0)
    ey = jnp.maximum(jnp.maximum(g_y1, q_y1) - jnp.minimum(g_y0, q_y0), 0.0)
    enclose = ex * ey
    giou = iou - (enclose - union) / enclose
    cost_ref[:] = (COST_CLASS * c_cls + COST_BBOX * l1) + COST_GIOU * (-giou)

    v_ref[:] = jnp.zeros((W, nc), f32)
    c4rv_ref[:] = jnp.full((W, nr), -1, i32)

    def init_col(t, carry):
        for w in range(W):
            r4c_s[w, t] = i32(-1)
        return carry

    jax.lax.fori_loop(0, nc + 1, init_col, 0)

    def init_row(t, carry):
        for w in range(W):
            u_s[w, t] = f32(0.0)
            c4r_s[w, t] = i32(-1)
        return carry

    jax.lax.fori_loop(0, nr + 1, init_row, 0)

    def outer(cur_row, carry):
        spc_ref[:] = jnp.full((W, nc), INF, f32)
        path_ref[:] = jnp.full((W, nc), -1, i32)
        rem_ref[:] = jnp.ones((W, nc), i32)

        def cond(st):
            alld = st[0][0]
            for w in range(1, W):
                alld = alld & st[0][w]
            return jnp.logical_not(alld)

        def body(st):
            done, i, mv_col, sink, k = st
            crow = jnp.concatenate(
                [cost_ref[pl.ds(w * nr + i[w], 1), :] for w in range(W)],
                axis=0)
            u_col = pack([u_s[w, i[w]] for w in range(W)], f32)
            i_col = pack(i, i32)
            act_col = pack([jnp.logical_not(d) for d in done], i32) != 0
            r = ((mv_col + crow) - u_col) - v_ref[:]
            rem = rem_ref[:] != 0
            spc = spc_ref[:]
            better = rem & (r < spc) & act_col
            spc = jnp.where(better, r, spc)
            spc_ref[:] = spc
            path_ref[:] = jnp.where(better, i_col, path_ref[:])
            masked = jnp.where(rem, spc, INF)
            mv2_col = jnp.min(masked, axis=1, keepdims=True)
            j_col = jnp.min(jnp.where(masked == mv2_col, iota_c, nc),
                            axis=1, keepdims=True)
            rem_ref[:] = jnp.where((iota_c == j_col) & act_col, 0,
                                   rem_ref[:])
            ndone, ni, nsink, nk = [], [], [], []
            for w in range(W):
                j_w = j_col[w, 0]
                mv2_w = mv2_col[w, 0]
                r4cj = r4c_s[w, j_w]
                unmatched = r4cj < 0
                ndone.append(done[w] | unmatched)
                nsink.append(jnp.where(done[w], sink[w],
                                       jnp.where(unmatched, j_w, sink[w])))
                ni.append(jnp.where(done[w] | unmatched, i[w], r4cj))
                visr_s[w, k[w]] = r4cj
                vism_s[w, k[w]] = mv2_w
                nk.append(jnp.where(done[w] | unmatched, k[w], k[w] + 1))
            nmv_col = jnp.where(pack(done, i32) != 0, mv_col, mv2_col)
            return tuple(ndone), tuple(ni), nmv_col, tuple(nsink), tuple(nk)

        init = (tuple(jnp.asarray(False) for _ in range(W)),
                tuple(i32(cur_row) for _ in range(W)),
                jnp.zeros((W, 1), f32),
                tuple(i32(-1) for _ in range(W)),
                tuple(i32(0) for _ in range(W)))
        _, _, mvf_col, sink, kf = jax.lax.while_loop(cond, body, init)

        for w in range(W):
            mvf_w = mvf_col[w, 0]
            u_s[w, cur_row] = u_s[w, cur_row] + mvf_w

            def dual(t, carry, w=w, mvf_w=mvf_w):
                row = visr_s[w, t]
                u_s[w, row] = u_s[w, row] + (mvf_w - vism_s[w, t])
                return carry

            jax.lax.fori_loop(0, kf[w], dual, 0)

        sc = rem_ref[:] == 0
        v_ref[:] = v_ref[:] - jnp.where(sc, mvf_col - spc_ref[:], 0.0)

        def acond(st):
            alld = st[0][0]
            for w in range(1, W):
                alld = alld & st[0][w]
            return jnp.logical_not(alld)

        def abody(st):
            done, j = st
            j_col = pack(j, i32)
            act_col = pack([jnp.logical_not(d) for d in done], i32) != 0
            pi_col = jnp.sum(jnp.where((iota_c == j_col) & act_col,
                                       path_ref[:], 0),
                             axis=1, keepdims=True)
            c4rv_ref[:] = jnp.where((iota_r == pi_col) & act_col,
                                    j_col, c4rv_ref[:])
            ndone, nj = [], []
            for w in range(W):
                pi = pi_col[w, 0]
                jw = jnp.where(done[w], nc, j[w])
                piw = jnp.where(done[w], nr, pi)
                r4c_s[w, jw] = pi
                jn = c4r_s[w, piw]
                c4r_s[w, piw] = j[w]
                ndone.append(done[w] | (pi == cur_row))
                nj.append(jnp.where(done[w], j[w], jn))
            return tuple(ndone), tuple(nj)

        ainit = (tuple(jnp.asarray(False) for _ in range(W)), sink)
        jax.lax.while_loop(acond, abody, ainit)
        return carry

    jax.lax.fori_loop(0, nr, outer, 0)

    iota_sub = jax.lax.broadcasted_iota(i32, (nr, 1), 0)
    for w in range(W):
        c4r = c4rv_ref[pl.ds(w, 1), :]
        c4r_col = c4r.reshape(nr, 1)
        rank = jnp.sum((c4r < c4r_col).astype(i32), axis=1, keepdims=True)
        oh = rank == iota_r1
        oj_ref[0, 0, w * nr:(w + 1) * nr] = jnp.sum(
            jnp.where(oh, iota_sub, 0), axis=0)
        oi_ref[0, 0, w * nr:(w + 1) * nr] = jnp.sum(
            jnp.where(oh, c4r_col, 0), axis=0)


def kernel(pred_logits, pred_boxes, tgt_labels, tgt_boxes):
    bs, nq, ncls = pred_logits.shape
    ngt = tgt_labels.shape[1]
    ncls_pad = ((ncls + 7) // 8) * 8
    ng = bs // W

    lg = jnp.zeros((ncls_pad, nq), jnp.float32).at[:ncls].set(pred_logits[0].T)
    bq = jnp.zeros((8, nq), jnp.float32).at[:4].set(pred_boxes[0].T)
    ids3 = tgt_labels.reshape(ng, W * ngt, 1).astype(jnp.int32)
    gt3 = tgt_boxes.reshape(ng, W * ngt, 4)

    body = functools.partial(_matcher_kernel, nr=ngt, nc=nq)
    scratch = [
        pltpu.VMEM((W * ngt, nq), jnp.float32),
        pltpu.VMEM((W, nq), jnp.float32),
        pltpu.VMEM((W, ngt), jnp.int32),
        pltpu.VMEM((W, nq), jnp.float32),
        pltpu.VMEM((W, nq), jnp.int32),
        pltpu.VMEM((W, nq), jnp.int32),
        pltpu.SMEM((W, ngt + 1), jnp.float32),
        pltpu.SMEM((W, nq + 1), jnp.int32),
        pltpu.SMEM((W, ngt + 1), jnp.int32),
        pltpu.SMEM((W, ngt + 2), jnp.int32),
        pltpu.SMEM((W, ngt + 2), jnp.float32),
    ]
    oi, oj = pl.pallas_call(
        body,
        grid=(ng,),
        in_specs=[
            pl.BlockSpec((ncls_pad, nq), lambda b: (0, 0)),
            pl.BlockSpec((8, nq), lambda b: (0, 0)),
            pl.BlockSpec((1, W * ngt, 1), lambda b: (b, 0, 0)),
            pl.BlockSpec((1, W * ngt, 4), lambda b: (b, 0, 0)),
        ],
        out_specs=[
            pl.BlockSpec((1, 1, W * ngt), lambda b: (b, 0, 0)),
            pl.BlockSpec((1, 1, W * ngt), lambda b: (b, 0, 0)),
        ],
        out_shape=[
            jax.ShapeDtypeStruct((ng, 1, W * ngt), jnp.int32),
            jax.ShapeDtypeStruct((ng, 1, W * ngt), jnp.int32),
        ],
        scratch_shapes=scratch,
        compiler_params=pltpu.CompilerParams(
            dimension_semantics=("parallel",),
        ),
    )(lg, bq, ids3, gt3)
    return oi.reshape(bs, ngt), oj.reshape(bs, ngt)


# speedup vs baseline: 2.6293x; 1.1833x over previous
import functools

import jax
import jax.numpy as jnp
from jax.experimental import pallas as pl
from jax.experimental.pallas import tpu as pltpu

COST_CLASS, COST_BBOX, COST_GIOU = 1.0, 5.0, 2.0
ALPHA, GAMMA = 0.25, 2.0
W = 16


def _matcher_kernel(lg_ref, bq_ref, ids_ref, gt_ref, oi_ref, oj_ref,
                    cost_ref, v_ref, c4rv_ref, spc_ref, path_ref, rem_ref,
                    u_s, r4c_s, c4r_s, visr_s, vism_s, *, nr, nc, W):
    f32 = jnp.float32
    i32 = jnp.int32
    INF = f32(jnp.inf)
    iota_c = jax.lax.broadcasted_iota(i32, (W, nc), 1)
    iota_r = jax.lax.broadcasted_iota(i32, (W, nr), 1)
    iota_r1 = jax.lax.broadcasted_iota(i32, (1, nr), 1)

    def pack(vals, dtype):
        return jnp.concatenate(
            [jnp.reshape(v, (1, 1)).astype(dtype) for v in vals], axis=0)

    p = jax.nn.sigmoid(lg_ref[:])
    neg = (1.0 - ALPHA) * (p * p) * (-jnp.log(1.0 - p))
    pos = ALPHA * ((1.0 - p) * (1.0 - p)) * (-jnp.log(p + 1e-8))
    diff = pos - neg
    q_cx = bq_ref[0:1, :]
    q_cy = bq_ref[1:2, :]
    q_w = bq_ref[2:3, :]
    q_h = bq_ref[3:4, :]
    q_x0 = q_cx - 0.5 * q_w
    q_y0 = q_cy - 0.5 * q_h
    q_x1 = q_cx + 0.5 * q_w
    q_y1 = q_cy + 0.5 * q_h
    area_q = (q_x1 - q_x0) * (q_y1 - q_y0)
    iota_cls = jax.lax.broadcasted_iota(i32, (nr, lg_ref.shape[0]), 1)

    def build_chunk(w, carry):
        sl = pl.ds(w * nr, nr)
        ids = ids_ref[0, sl, :]
        onehot = (iota_cls == ids).astype(f32)
        c_cls = jnp.dot(onehot, diff, preferred_element_type=f32,
                        precision=jax.lax.Precision.HIGHEST)

        g_cx = gt_ref[0, sl, 0:1]
        g_cy = gt_ref[0, sl, 1:2]
        g_w = gt_ref[0, sl, 2:3]
        g_h = gt_ref[0, sl, 3:4]

        l1 = ((jnp.abs(g_cx - q_cx) + jnp.abs(g_cy - q_cy))
              + jnp.abs(g_w - q_w)) + jnp.abs(g_h - q_h)

        g_x0 = g_cx - 0.5 * g_w
        g_y0 = g_cy - 0.5 * g_h
        g_x1 = g_cx + 0.5 * g_w
        g_y1 = g_cy + 0.5 * g_h
        area_g = (g_x1 - g_x0) * (g_y1 - g_y0)
        whx = jnp.maximum(
            jnp.minimum(g_x1, q_x1) - jnp.maximum(g_x0, q_x0), 0.0)
        why = jnp.maximum(
            jnp.minimum(g_y1, q_y1) - jnp.maximum(g_y0, q_y0), 0.0)
        inter = whx * why
        union = area_q + area_g - inter
        iou = inter / union
        ex = jnp.maximum(
            jnp.maximum(g_x1, q_x1) - jnp.minimum(g_x0, q_x0), 0.0)
        ey = jnp.maximum(
            jnp.maximum(g_y1, q_y1) - jnp.minimum(g_y0, q_y0), 0.0)
        enclose = ex * ey
        giou = iou - (enclose - union) / enclose
        cost_ref[sl, :] = ((COST_CLASS * c_cls + COST_BBOX * l1)
                           + COST_GIOU * (-giou))
        return carry

    jax.lax.fori_loop(0, W, build_chunk, 0)

    v_ref[:] = jnp.zeros((W, nc), f32)
    c4rv_ref[:] = jnp.full((W, nr), -1, i32)

    def init_col(t, carry):
        for w in range(W):
            r4c_s[w, t] = i32(-1)
        return carry

    jax.lax.fori_loop(0, nc + 1, init_col, 0)

    def init_row(t, carry):
        for w in range(W):
            u_s[w, t] = f32(0.0)
            c4r_s[w, t] = i32(-1)
        return carry

    jax.lax.fori_loop(0, nr + 1, init_row, 0)

    def outer(cur_row, carry):
        spc_ref[:] = jnp.full((W, nc), INF, f32)
        path_ref[:] = jnp.full((W, nc), -1, i32)
        rem_ref[:] = jnp.ones((W, nc), i32)

        def cond(st):
            alld = st[0][0]
            for w in range(1, W):
                alld = alld & st[0][w]
            return jnp.logical_not(alld)

        def body(st):
            done, i, mv_col, sink, k = st
            crow = jnp.concatenate(
                [cost_ref[pl.ds(w * nr + i[w], 1), :] for w in range(W)],
                axis=0)
            u_col = pack([u_s[w, i[w]] for w in range(W)], f32)
            i_col = pack(i, i32)
            act_col = pack([jnp.logical_not(d) for d in done], i32) != 0
            r = ((mv_col + crow) - u_col) - v_ref[:]
            rem = rem_ref[:] != 0
            spc = spc_ref[:]
            better = rem & (r < spc) & act_col
            spc = jnp.where(better, r, spc)
            spc_ref[:] = spc
            path_ref[:] = jnp.where(better, i_col, path_ref[:])
            masked = jnp.where(rem, spc, INF)
            mv2_col = jnp.min(masked, axis=1, keepdims=True)
            j_col = jnp.min(jnp.where(masked == mv2_col, iota_c, nc),
                            axis=1, keepdims=True)
            rem_ref[:] = jnp.where((iota_c == j_col) & act_col, 0,
                                   rem_ref[:])
            ndone, ni, nsink, nk = [], [], [], []
            for w in range(W):
                j_w = j_col[w, 0]
                mv2_w = mv2_col[w, 0]
                r4cj = r4c_s[w, j_w]
                unmatched = r4cj < 0
                ndone.append(done[w] | unmatched)
                nsink.append(jnp.where(done[w], sink[w],
                                       jnp.where(unmatched, j_w, sink[w])))
                ni.append(jnp.where(done[w] | unmatched, i[w], r4cj))
                visr_s[w, k[w]] = r4cj
                vism_s[w, k[w]] = mv2_w
                nk.append(jnp.where(done[w] | unmatched, k[w], k[w] + 1))
            nmv_col = jnp.where(pack(done, i32) != 0, mv_col, mv2_col)
            return tuple(ndone), tuple(ni), nmv_col, tuple(nsink), tuple(nk)

        init = (tuple(jnp.asarray(False) for _ in range(W)),
                tuple(i32(cur_row) for _ in range(W)),
                jnp.zeros((W, 1), f32),
                tuple(i32(-1) for _ in range(W)),
                tuple(i32(0) for _ in range(W)))
        _, _, mvf_col, sink, kf = jax.lax.while_loop(cond, body, init)

        for w in range(W):
            mvf_w = mvf_col[w, 0]
            u_s[w, cur_row] = u_s[w, cur_row] + mvf_w

            def dual(t, carry, w=w, mvf_w=mvf_w):
                row = visr_s[w, t]
                u_s[w, row] = u_s[w, row] + (mvf_w - vism_s[w, t])
                return carry

            jax.lax.fori_loop(0, kf[w], dual, 0)

        sc = rem_ref[:] == 0
        v_ref[:] = v_ref[:] - jnp.where(sc, mvf_col - spc_ref[:], 0.0)

        def acond(st):
            alld = st[0][0]
            for w in range(1, W):
                alld = alld & st[0][w]
            return jnp.logical_not(alld)

        def abody(st):
            done, j = st
            j_col = pack(j, i32)
            act_col = pack([jnp.logical_not(d) for d in done], i32) != 0
            pi_col = jnp.sum(jnp.where((iota_c == j_col) & act_col,
                                       path_ref[:], 0),
                             axis=1, keepdims=True)
            c4rv_ref[:] = jnp.where((iota_r == pi_col) & act_col,
                                    j_col, c4rv_ref[:])
            ndone, nj = [], []
            for w in range(W):
                pi = pi_col[w, 0]
                jw = jnp.where(done[w], nc, j[w])
                piw = jnp.where(done[w], nr, pi)
                r4c_s[w, jw] = pi
                jn = c4r_s[w, piw]
                c4r_s[w, piw] = j[w]
                ndone.append(done[w] | (pi == cur_row))
                nj.append(jnp.where(done[w], j[w], jn))
            return tuple(ndone), tuple(nj)

        ainit = (tuple(jnp.asarray(False) for _ in range(W)), sink)
        jax.lax.while_loop(acond, abody, ainit)
        return carry

    jax.lax.fori_loop(0, nr, outer, 0)

    iota_sub = jax.lax.broadcasted_iota(i32, (nr, 1), 0)
    for w in range(W):
        c4r = c4rv_ref[pl.ds(w, 1), :]
        c4r_col = c4r.reshape(nr, 1)
        rank = jnp.sum((c4r < c4r_col).astype(i32), axis=1, keepdims=True)
        oh = rank == iota_r1
        oj_ref[0, 0, w * nr:(w + 1) * nr] = jnp.sum(
            jnp.where(oh, iota_sub, 0), axis=0)
        oi_ref[0, 0, w * nr:(w + 1) * nr] = jnp.sum(
            jnp.where(oh, c4r_col, 0), axis=0)


def kernel(pred_logits, pred_boxes, tgt_labels, tgt_boxes):
    bs, nq, ncls = pred_logits.shape
    ngt = tgt_labels.shape[1]
    ncls_pad = ((ncls + 7) // 8) * 8
    W_eff = min(W, bs)
    ng = bs // W_eff

    lg = jnp.zeros((ncls_pad, nq), jnp.float32).at[:ncls].set(pred_logits[0].T)
    bq = jnp.zeros((8, nq), jnp.float32).at[:4].set(pred_boxes[0].T)
    ids3 = tgt_labels.reshape(ng, W_eff * ngt, 1).astype(jnp.int32)
    gt3 = tgt_boxes.reshape(ng, W_eff * ngt, 4)

    body = functools.partial(_matcher_kernel, nr=ngt, nc=nq, W=W_eff)
    scratch = [
        pltpu.VMEM((W_eff * ngt, nq), jnp.float32),
        pltpu.VMEM((W_eff, nq), jnp.float32),
        pltpu.VMEM((W_eff, ngt), jnp.int32),
        pltpu.VMEM((W_eff, nq), jnp.float32),
        pltpu.VMEM((W_eff, nq), jnp.int32),
        pltpu.VMEM((W_eff, nq), jnp.int32),
        pltpu.SMEM((W_eff, ngt + 1), jnp.float32),
        pltpu.SMEM((W_eff, nq + 1), jnp.int32),
        pltpu.SMEM((W_eff, ngt + 1), jnp.int32),
        pltpu.SMEM((W_eff, ngt + 2), jnp.int32),
        pltpu.SMEM((W_eff, ngt + 2), jnp.float32),
    ]
    oi, oj = pl.pallas_call(
        body,
        grid=(ng,),
        in_specs=[
            pl.BlockSpec((ncls_pad, nq), lambda b: (0, 0)),
            pl.BlockSpec((8, nq), lambda b: (0, 0)),
            pl.BlockSpec((1, W_eff * ngt, 1), lambda b: (b, 0, 0)),
            pl.BlockSpec((1, W_eff * ngt, 4), lambda b: (b, 0, 0)),
        ],
        out_specs=[
            pl.BlockSpec((1, 1, W_eff * ngt), lambda b: (b, 0, 0)),
            pl.BlockSpec((1, 1, W_eff * ngt), lambda b: (b, 0, 0)),
        ],
        out_shape=[
            jax.ShapeDtypeStruct((ng, 1, W_eff * ngt), jnp.int32),
            jax.ShapeDtypeStruct((ng, 1, W_eff * ngt), jnp.int32),
        ],
        scratch_shapes=scratch,
        compiler_params=pltpu.CompilerParams(
            dimension_semantics=("parallel",),
        ),
    )(lg, bq, ids3, gt3)
    return oi.reshape(bs, ngt), oj.reshape(bs, ngt)


# speedup vs baseline: 4.3246x; 1.6448x over previous
import functools

import jax
import jax.numpy as jnp
from jax.experimental import pallas as pl
from jax.experimental.pallas import tpu as pltpu

COST_CLASS, COST_BBOX, COST_GIOU = 1.0, 5.0, 2.0
ALPHA, GAMMA = 0.25, 2.0
W = 16


def _matcher_kernel(lg_ref, bq_ref, ids_ref, gt_ref, oi_ref, oj_ref,
                    cost_ref, v_ref, c4rv_ref, spc_ref, path_ref, rem_ref,
                    r4cv_ref, u_ref, sr_ref, mvr_ref, *, nr, nc, W):
    f32 = jnp.float32
    i32 = jnp.int32
    INF = f32(jnp.inf)
    iota_c = jax.lax.broadcasted_iota(i32, (W, nc), 1)
    iota_r = jax.lax.broadcasted_iota(i32, (W, nr), 1)
    iota_r1 = jax.lax.broadcasted_iota(i32, (1, nr), 1)

    def pack(vals, dtype):
        return jnp.concatenate(
            [jnp.reshape(v, (1, 1)).astype(dtype) for v in vals], axis=0)

    p = jax.nn.sigmoid(lg_ref[:])
    neg = (1.0 - ALPHA) * (p * p) * (-jnp.log(1.0 - p))
    pos = ALPHA * ((1.0 - p) * (1.0 - p)) * (-jnp.log(p + 1e-8))
    diff = pos - neg
    q_cx = bq_ref[0:1, :]
    q_cy = bq_ref[1:2, :]
    q_w = bq_ref[2:3, :]
    q_h = bq_ref[3:4, :]
    q_x0 = q_cx - 0.5 * q_w
    q_y0 = q_cy - 0.5 * q_h
    q_x1 = q_cx + 0.5 * q_w
    q_y1 = q_cy + 0.5 * q_h
    area_q = (q_x1 - q_x0) * (q_y1 - q_y0)
    iota_cls = jax.lax.broadcasted_iota(i32, (nr, lg_ref.shape[0]), 1)

    def build_chunk(w, carry):
        sl = pl.ds(w * nr, nr)
        ids = ids_ref[0, sl, :]
        onehot = (iota_cls == ids).astype(f32)
        c_cls = jnp.dot(onehot, diff, preferred_element_type=f32,
                        precision=jax.lax.Precision.HIGHEST)

        g_cx = gt_ref[0, sl, 0:1]
        g_cy = gt_ref[0, sl, 1:2]
        g_w = gt_ref[0, sl, 2:3]
        g_h = gt_ref[0, sl, 3:4]

        l1 = ((jnp.abs(g_cx - q_cx) + jnp.abs(g_cy - q_cy))
              + jnp.abs(g_w - q_w)) + jnp.abs(g_h - q_h)

        g_x0 = g_cx - 0.5 * g_w
        g_y0 = g_cy - 0.5 * g_h
        g_x1 = g_cx + 0.5 * g_w
        g_y1 = g_cy + 0.5 * g_h
        area_g = (g_x1 - g_x0) * (g_y1 - g_y0)
        whx = jnp.maximum(
            jnp.minimum(g_x1, q_x1) - jnp.maximum(g_x0, q_x0), 0.0)
        why = jnp.maximum(
            jnp.minimum(g_y1, q_y1) - jnp.maximum(g_y0, q_y0), 0.0)
        inter = whx * why
        union = area_q + area_g - inter
        iou = inter / union
        ex = jnp.maximum(
            jnp.maximum(g_x1, q_x1) - jnp.minimum(g_x0, q_x0), 0.0)
        ey = jnp.maximum(
            jnp.maximum(g_y1, q_y1) - jnp.minimum(g_y0, q_y0), 0.0)
        enclose = ex * ey
        giou = iou - (enclose - union) / enclose
        cost_ref[sl, :] = ((COST_CLASS * c_cls + COST_BBOX * l1)
                           + COST_GIOU * (-giou))
        return carry

    jax.lax.fori_loop(0, W, build_chunk, 0)

    v_ref[:] = jnp.zeros((W, nc), f32)
    c4rv_ref[:] = jnp.full((W, nr), -1, i32)
    r4cv_ref[:] = jnp.full((W, nc), -1, i32)
    u_ref[:] = jnp.zeros((W, nr), f32)

    def outer(cur_row, carry):
        spc_ref[:] = jnp.full((W, nc), INF, f32)
        path_ref[:] = jnp.full((W, nc), -1, i32)
        rem_ref[:] = jnp.ones((W, nc), i32)
        sr_ref[:] = jnp.zeros((W, nr), i32)

        def cond(st):
            return jnp.any(st[0] == 0)

        def body(st):
            done_i, i, mv_col, sink_col = st
            done_col = done_i != 0
            crow = jnp.concatenate(
                [cost_ref[pl.ds(w * nr + i[w], 1), :] for w in range(W)],
                axis=0)
            i_col = pack(i, i32)
            act_col = jnp.logical_not(done_col)
            u_col = jnp.sum(jnp.where(iota_r == i_col, u_ref[:], 0.0),
                            axis=1, keepdims=True)
            sr_ref[:] = jnp.where((iota_r == i_col) & act_col, 1, sr_ref[:])
            r = ((mv_col + crow) - u_col) - v_ref[:]
            rem = rem_ref[:] != 0
            spc = spc_ref[:]
            better = rem & (r < spc) & act_col
            spc = jnp.where(better, r, spc)
            spc_ref[:] = spc
            path_ref[:] = jnp.where(better, i_col, path_ref[:])
            masked = jnp.where(rem, spc, INF)
            mv2_col = jnp.min(masked, axis=1, keepdims=True)
            j_col = jnp.min(jnp.where(masked == mv2_col, iota_c, nc),
                            axis=1, keepdims=True)
            rem_ref[:] = jnp.where((iota_c == j_col) & act_col, 0,
                                   rem_ref[:])
            r4cj_col = jnp.sum(jnp.where(iota_c == j_col, r4cv_ref[:], 0),
                               axis=1, keepdims=True)
            unm_col = r4cj_col < 0
            mvr_ref[:] = jnp.where((iota_r == r4cj_col) & act_col,
                                   mv2_col, mvr_ref[:])
            ndone_col = done_col | unm_col
            nmv_col = jnp.where(done_col, mv_col, mv2_col)
            nsink_col = jnp.where(act_col & unm_col, j_col, sink_col)
            i_next_col = jnp.where(ndone_col, i_col, r4cj_col)
            ni = tuple(i_next_col[w, 0] for w in range(W))
            return ndone_col.astype(jnp.int32), ni, nmv_col, nsink_col

        init = (jnp.zeros((W, 1), jnp.int32),
                tuple(i32(cur_row) for _ in range(W)),
                jnp.zeros((W, 1), f32),
                jnp.full((W, 1), -1, i32))
        _, _, mvf_col, sink_col = jax.lax.while_loop(cond, body, init)

        sr = sr_ref[:] != 0
        u_ref[:] = u_ref[:] + jnp.where(
            sr, jnp.where(iota_r == cur_row, mvf_col, mvf_col - mvr_ref[:]),
            0.0)
        sc = rem_ref[:] == 0
        v_ref[:] = v_ref[:] - jnp.where(sc, mvf_col - spc_ref[:], 0.0)

        def acond(st):
            return jnp.any(st[0] == 0)

        def abody(st):
            done_i, j_col = st
            done_col = done_i != 0
            act_col = jnp.logical_not(done_col)
            jmask = iota_c == j_col
            pi_col = jnp.sum(jnp.where(jmask & act_col, path_ref[:], 0),
                             axis=1, keepdims=True)
            pimask = iota_r == pi_col
            r4cv_ref[:] = jnp.where(jmask & act_col, pi_col, r4cv_ref[:])
            jn_col = jnp.sum(jnp.where(pimask & act_col, c4rv_ref[:], 0),
                             axis=1, keepdims=True)
            c4rv_ref[:] = jnp.where(pimask & act_col, j_col, c4rv_ref[:])
            ndone_col = done_col | (pi_col == cur_row)
            nj_col = jnp.where(done_col, j_col, jn_col)
            return ndone_col.astype(jnp.int32), nj_col

        ainit = (jnp.zeros((W, 1), jnp.int32), sink_col)
        jax.lax.while_loop(acond, abody, ainit)
        return carry

    jax.lax.fori_loop(0, nr, outer, 0)

    iota_sub = jax.lax.broadcasted_iota(i32, (nr, 1), 0)
    for w in range(W):
        c4r = c4rv_ref[pl.ds(w, 1), :]
        c4r_col = c4r.reshape(nr, 1)
        rank = jnp.sum((c4r < c4r_col).astype(i32), axis=1, keepdims=True)
        oh = rank == iota_r1
        oj_ref[0, 0, w * nr:(w + 1) * nr] = jnp.sum(
            jnp.where(oh, iota_sub, 0), axis=0)
        oi_ref[0, 0, w * nr:(w + 1) * nr] = jnp.sum(
            jnp.where(oh, c4r_col, 0), axis=0)


def kernel(pred_logits, pred_boxes, tgt_labels, tgt_boxes):
    bs, nq, ncls = pred_logits.shape
    ngt = tgt_labels.shape[1]
    ncls_pad = ((ncls + 7) // 8) * 8
    W_eff = min(W, bs)
    ng = bs // W_eff

    lg = jnp.zeros((ncls_pad, nq), jnp.float32).at[:ncls].set(pred_logits[0].T)
    bq = jnp.zeros((8, nq), jnp.float32).at[:4].set(pred_boxes[0].T)
    ids3 = tgt_labels.reshape(ng, W_eff * ngt, 1).astype(jnp.int32)
    gt3 = tgt_boxes.reshape(ng, W_eff * ngt, 4)

    body = functools.partial(_matcher_kernel, nr=ngt, nc=nq, W=W_eff)
    scratch = [
        pltpu.VMEM((W_eff * ngt, nq), jnp.float32),
        pltpu.VMEM((W_eff, nq), jnp.float32),
        pltpu.VMEM((W_eff, ngt), jnp.int32),
        pltpu.VMEM((W_eff, nq), jnp.float32),
        pltpu.VMEM((W_eff, nq), jnp.int32),
        pltpu.VMEM((W_eff, nq), jnp.int32),
        pltpu.VMEM((W_eff, nq), jnp.int32),
        pltpu.VMEM((W_eff, ngt), jnp.float32),
        pltpu.VMEM((W_eff, ngt), jnp.int32),
        pltpu.VMEM((W_eff, ngt), jnp.float32),
    ]
    oi, oj = pl.pallas_call(
        body,
        grid=(ng,),
        in_specs=[
            pl.BlockSpec((ncls_pad, nq), lambda b: (0, 0)),
            pl.BlockSpec((8, nq), lambda b: (0, 0)),
            pl.BlockSpec((1, W_eff * ngt, 1), lambda b: (b, 0, 0)),
            pl.BlockSpec((1, W_eff * ngt, 4), lambda b: (b, 0, 0)),
        ],
        out_specs=[
            pl.BlockSpec((1, 1, W_eff * ngt), lambda b: (b, 0, 0)),
            pl.BlockSpec((1, 1, W_eff * ngt), lambda b: (b, 0, 0)),
        ],
        out_shape=[
            jax.ShapeDtypeStruct((ng, 1, W_eff * ngt), jnp.int32),
            jax.ShapeDtypeStruct((ng, 1, W_eff * ngt), jnp.int32),
        ],
        scratch_shapes=scratch,
        compiler_params=pltpu.CompilerParams(
            dimension_semantics=("parallel",),
        ),
    )(lg, bq, ids3, gt3)
    return oi.reshape(bs, ngt), oj.reshape(bs, ngt)
